# baseline (speedup 1.0000x reference)
import functools

import jax
import jax.numpy as jnp
from jax import lax
from jax.experimental import pallas as pl
from jax.experimental.pallas import tpu as pltpu
from jax.experimental.pallas import tpu_sc as plsc

BATCH = 4
SEQ = 8192
DIM = 1024
LANES = 16
NUM_WORKERS = 32
ROWS_PER_WORKER = (BATCH * SEQ) // NUM_WORKERS
CHUNKS_PER_ROW = SEQ // ROWS_PER_WORKER
R = 16
NBUF = 4
NCHUNK = ROWS_PER_WORKER // R
VECS = ROWS_PER_WORKER // LANES


def _sc_body(mask_hbm, pkv_hbm, table_hbm, out_hbm,
             mask_v, idx_v, pkv_v,
             buf0, buf1, buf2, buf3, s0, s1, s2, s3):
    bufs = (buf0, buf1, buf2, buf3)
    sems = (s0, s1, s2, s3)

    wid = lax.axis_index("s") * 2 + lax.axis_index("c")
    b = wid // CHUNKS_PER_ROW
    c = wid % CHUNKS_PER_ROW
    row0 = wid * ROWS_PER_WORKER

    pltpu.sync_copy(mask_hbm.at[pl.ds(b * SEQ, SEQ)], mask_v)
    pltpu.sync_copy(pkv_hbm, pkv_v)
    pkv_vec = pkv_v[...]

    def _pre(i, acc):
        return acc + mask_v[pl.ds(i * LANES, LANES)]
    acc = lax.fori_loop(0, c * (1024 // LANES), _pre,
                        jnp.zeros((LANES,), jnp.int32))
    offset = jnp.sum(acc)

    base = c * 1024

    def _cum(j, carry):
        mvec = mask_v[pl.ds(base + j * LANES, LANES)]
        cum = plsc.cumsum(mvec)
        idx_v[j, :] = (cum + carry + pkv_vec) * mvec + 1
        return carry + jnp.sum(mvec)

    lax.fori_loop(0, VECS, _cum, offset)

    for k in range(NBUF):
        pltpu.async_copy(table_hbm.at[idx_v.at[k]], bufs[k], sems[k])

    def _step(t, carry):
        for k in range(NBUF):
            j = t * NBUF + k
            pltpu.make_async_copy(table_hbm.at[idx_v.at[j]], bufs[k],
                                  sems[k]).wait()
            pltpu.sync_copy(bufs[k], out_hbm.at[pl.ds(row0 + j * R, R)])

            @pl.when(j + NBUF < NCHUNK)
            def _():
                pltpu.async_copy(table_hbm.at[idx_v.at[j + NBUF]],
                                 bufs[k], sems[k])
        return carry

    lax.fori_loop(0, NCHUNK // NBUF, _step, 0)


@functools.partial(jax.jit, static_argnames=())
def _sc_embed(mask_flat, pkv_arr, weights):
    mesh = plsc.VectorSubcoreMesh(core_axis_name="c", subcore_axis_name="s")
    f = functools.partial(
        pl.kernel,
        mesh=mesh,
        compiler_params=pltpu.CompilerParams(needs_layout_passes=False),
        out_type=jax.ShapeDtypeStruct((BATCH * SEQ, DIM), jnp.float32),
        scratch_types=[
            pltpu.VMEM((SEQ,), jnp.int32),
            pltpu.VMEM((VECS, LANES), jnp.int32),
            pltpu.VMEM((LANES,), jnp.int32),
            pltpu.VMEM((R, DIM), jnp.float32),
            pltpu.VMEM((R, DIM), jnp.float32),
            pltpu.VMEM((R, DIM), jnp.float32),
            pltpu.VMEM((R, DIM), jnp.float32),
            pltpu.SemaphoreType.DMA,
            pltpu.SemaphoreType.DMA,
            pltpu.SemaphoreType.DMA,
            pltpu.SemaphoreType.DMA,
        ],
    )(_sc_body)
    return f(mask_flat, pkv_arr, weights)


def kernel(input_ids, attention_mask, past_key_values_length, weights):
    del input_ids
    mask_flat = attention_mask.reshape(-1).astype(jnp.int32)
    pkv_arr = jnp.full((LANES,), past_key_values_length, jnp.int32)
    out = _sc_embed(mask_flat, pkv_arr, weights)
    return out.reshape(BATCH, SEQ, weights.shape[-1])
